# Initial kernel scaffold; baseline (speedup 1.0000x reference)
import jax
import jax.numpy as jnp
from jax.experimental import pallas as pl


def kernel(x, conv1_w, conv1_b, conv2_w, conv2_b, fc1_w, fc1_b, fc2_w, fc2_b):
    raise NotImplementedError("write your pallas kernel here")


# speedup vs baseline: 13.0461x; 13.0461x over previous
import numpy as np
import jax
import jax.numpy as jnp
from jax import lax
from jax.experimental import pallas as pl
from jax.experimental.pallas import tpu as pltpu

LANE = 128
H = W = 28
HC = 26
HP = 13
HC2 = 11
HP2 = 5
C1 = 16
C2 = 32
XW = 32

K1 = 128
N1H = 256
K2 = 256
N2 = 384
KF = 320


def _prep_weights(c1w, c1b, c2w, c2b, f1w, f1b, f2w, f2b):
    f32, bf16 = jnp.float32, jnp.bfloat16

    w1k = c1w.reshape(C1, 9).T.astype(f32)
    halves = []
    for half in range(2):
        acc = jnp.zeros((K1, N1H), f32)
        for di in range(3):
            for dj in range(3):
                k = di * 3 + dj
                E = np.zeros((HC, HP), np.float32)
                for m in range(HP):
                    E[2 * m + half, m] = 1.0
                blk = (jnp.asarray(E)[:, :, None] * w1k[k][None, None, :])
                blk = blk.reshape(HC, HP * C1)
                acc = acc + jnp.pad(
                    blk, ((di * XW + dj, K1 - HC - di * XW - dj),
                          (0, N1H - HP * C1)))
        halves.append(acc)
    w1 = jnp.concatenate(halves, axis=1).astype(bf16)

    w2v = jnp.transpose(c2w, (2, 3, 1, 0)).astype(f32)
    eye11 = jnp.asarray(np.eye(HC2, dtype=np.float32))
    w2_rows = []
    for di2 in range(3):
        acc = jnp.zeros((K2, N2), f32)
        for dj2 in range(3):
            T = (eye11[:, None, :, None] *
                 w2v[di2, dj2][None, :, None, :])
            blk = T.reshape(HC2 * C1, HC2 * C2)
            acc = acc + jnp.pad(
                blk, ((dj2 * C1, K2 - HC2 * C1 - dj2 * C1),
                      (0, N2 - HC2 * C2)))
        w2_rows.append(acc)
    w2 = jnp.stack(w2_rows).astype(bf16)

    f1r = f1w.reshape(f1w.shape[0], C2, HP2, HP2).astype(f32)
    f1_rows = []
    for pi2 in range(HP2):
        A = jnp.transpose(f1r[:, :, pi2, :], (2, 1, 0))
        A = jnp.pad(A, ((0, 0), (0, 0), (0, LANE - A.shape[2])))
        Z = jnp.stack([A, jnp.zeros_like(A)], axis=1)
        f1_rows.append(Z.reshape(HP2 * 2 * C2, LANE)[:KF])
    f1 = jnp.stack(f1_rows).astype(bf16)

    f2 = jnp.pad(f2w.T.astype(f32),
                 ((0, LANE - f2w.shape[1]), (0, LANE - f2w.shape[0])))

    b1 = jnp.tile(c1b.astype(f32), HP)
    b1 = jnp.pad(b1, (0, N1H - b1.shape[0])).reshape(1, N1H)
    b2 = jnp.tile(c2b.astype(f32), HC2)
    b2 = jnp.pad(b2, (0, N2 - b2.shape[0])).reshape(1, N2)
    fb1 = jnp.pad(f1b.astype(f32), (0, LANE - f1b.shape[0])).reshape(1, LANE)
    fb2 = jnp.pad(f2b.astype(f32), (0, LANE - f2b.shape[0])).reshape(1, LANE)
    return w1, w2, f1, f2, b1, b2, fb1, fb2


def kernel(x, conv1_w, conv1_b, conv2_w, conv2_b, fc1_w, fc1_b, fc2_w, fc2_b):
    n = x.shape[0]
    num_outputs = fc2_w.shape[0]
    f32, bf16 = jnp.float32, jnp.bfloat16

    B = next((b for b in (128, 64, 32, 16, 8) if n % b == 0), n)
    R = B * H
    R2 = B * (H // 2)
    M2 = R2 - 2

    xr = jnp.pad(x.reshape(n * H, W), ((0, 0), (0, XW - W)))
    w1, w2, f1, f2, b1, b2, fb1, fb2 = _prep_weights(
        conv1_w, conv1_b, conv2_w, conv2_b, fc1_w, fc1_b, fc2_w, fc2_b)

    def body(x_ref, w1_ref, w2_ref, f1_ref, f2_ref,
             b1_ref, b2_ref, fb1_ref, fb2_ref, out_ref,
             lhs1, ye0, ye1, yo0, yo1, p1, sa, sb, sc):
        y1 = (ye0, ye1, yo0, yo1)
        s2 = (sa, sb, sc)
        lhs1[...] = jnp.zeros_like(lhs1)
        for di in range(3):
            lhs1[pl.ds(0, R - di), di * XW: di * XW + XW] = (
                x_ref[pl.ds(di, R - di), :].astype(bf16))

        lhs = lhs1[...]
        for q in range(4):
            y1[q][...] = jnp.dot(lhs, w1_ref[:, q * 128:(q + 1) * 128],
                                 preferred_element_type=f32)

        for q in range(2):
            ye, yo = y1[q], y1[q + 2]
            m = jnp.maximum(
                jnp.maximum(ye[pl.ds(0, R2, 2), :], ye[pl.ds(1, R2, 2), :]),
                jnp.maximum(yo[pl.ds(0, R2, 2), :], yo[pl.ds(1, R2, 2), :]))
            p1[:, q * 128:(q + 1) * 128] = jnp.maximum(
                m + b1_ref[:, q * 128:(q + 1) * 128], 0.0).astype(bf16)

        acc = None
        for di2 in range(3):
            lhs2 = p1[pl.ds(di2, M2), :]
            prod = jnp.dot(lhs2, w2_ref[di2], preferred_element_type=f32)
            acc = prod if acc is None else acc + prod
        acc = jnp.maximum(acc + b2_ref[...], 0.0)
        for q in range(3):
            s2[q][pl.ds(0, M2), :] = acc[:, q * 128:(q + 1) * 128]

        hacc = None
        for pi2 in range(HP2):
            vm = jnp.concatenate(
                [jnp.maximum(s2[q][pl.ds(2 * pi2, B, 14), :],
                             s2[q][pl.ds(2 * pi2 + 1, B, 14), :])
                 for q in range(3)], axis=1)
            vmm = jnp.maximum(vm[:, 0:KF], vm[:, C2:KF + C2])
            prod = jnp.dot(vmm.astype(bf16), f1_ref[pi2],
                           preferred_element_type=f32)
            hacc = prod if hacc is None else hacc + prod
        hidden = jnp.maximum(hacc + fb1_ref[...], 0.0)

        logits = jnp.dot(hidden, f2_ref[...],
                         preferred_element_type=f32) + fb2_ref[...]
        lane = lax.broadcasted_iota(jnp.int32, (B, LANE), 1)
        logits = jnp.where(lane < num_outputs, logits, -1e30)
        mx = jnp.max(logits, axis=-1, keepdims=True)
        lse = jnp.log(jnp.sum(jnp.exp(logits - mx), axis=-1, keepdims=True))
        out_ref[...] = logits - mx - lse

    grid_spec = pltpu.PrefetchScalarGridSpec(
        num_scalar_prefetch=0,
        grid=(n // B,),
        in_specs=[
            pl.BlockSpec((R, XW), lambda g: (g, 0)),
            pl.BlockSpec((K1, 2 * N1H), lambda g: (0, 0)),
            pl.BlockSpec((3, K2, N2), lambda g: (0, 0, 0)),
            pl.BlockSpec((HP2, KF, LANE), lambda g: (0, 0, 0)),
            pl.BlockSpec((LANE, LANE), lambda g: (0, 0)),
            pl.BlockSpec((1, N1H), lambda g: (0, 0)),
            pl.BlockSpec((1, N2), lambda g: (0, 0)),
            pl.BlockSpec((1, LANE), lambda g: (0, 0)),
            pl.BlockSpec((1, LANE), lambda g: (0, 0)),
        ],
        out_specs=pl.BlockSpec((B, LANE), lambda g: (g, 0)),
        scratch_shapes=[
            pltpu.VMEM((R, K1), bf16),
            pltpu.VMEM((R, 128), f32),
            pltpu.VMEM((R, 128), f32),
            pltpu.VMEM((R, 128), f32),
            pltpu.VMEM((R, 128), f32),
            pltpu.VMEM((R2, K2), bf16),
            pltpu.VMEM((R2, 128), f32),
            pltpu.VMEM((R2, 128), f32),
            pltpu.VMEM((R2, 128), f32),
        ],
    )

    out = pl.pallas_call(
        body,
        out_shape=jax.ShapeDtypeStruct((n, LANE), f32),
        grid_spec=grid_spec,
        compiler_params=pltpu.CompilerParams(
            dimension_semantics=("parallel",)),
    )(xr, w1, w2, f1, f2, b1, b2, fb1, fb2)
    return out[:, :num_outputs]


# speedup vs baseline: 15.1927x; 1.1645x over previous
import numpy as np
import jax
import jax.numpy as jnp
from jax import lax
from jax.experimental import pallas as pl
from jax.experimental.pallas import tpu as pltpu

LANE = 128
H = W = 28
HC = 26
HP = 13
HC2 = 11
HP2 = 5
C1 = 16
C2 = 32
XW = 28

K1 = 128
N1H = 256
K2 = 256
N2 = 384
KF = 320


def _prep_weights(c1w, c1b, c2w, c2b, f1w, f1b, f2w, f2b):
    f32, bf16 = jnp.float32, jnp.bfloat16

    w1k = c1w.reshape(C1, 9).T.astype(f32)
    halves = []
    for half in range(2):
        acc = jnp.zeros((K1, N1H), f32)
        for di in range(3):
            for dj in range(3):
                k = di * 3 + dj
                E = np.zeros((HC, HP), np.float32)
                for m in range(HP):
                    E[2 * m + half, m] = 1.0
                blk = (jnp.asarray(E)[:, :, None] * w1k[k][None, None, :])
                blk = blk.reshape(HC, HP * C1)
                acc = acc + jnp.pad(
                    blk, ((di * XW + dj, K1 - HC - di * XW - dj),
                          (0, N1H - HP * C1)))
        halves.append(acc)
    w1 = jnp.concatenate(halves, axis=1).astype(bf16)

    w2v = jnp.transpose(c2w, (2, 3, 1, 0)).astype(f32)
    eye11 = jnp.asarray(np.eye(HC2, dtype=np.float32))
    w2_rows = []
    for di2 in range(3):
        acc = jnp.zeros((K2, N2), f32)
        for dj2 in range(3):
            T = (eye11[:, None, :, None] *
                 w2v[di2, dj2][None, :, None, :])
            blk = T.reshape(HC2 * C1, HC2 * C2)
            acc = acc + jnp.pad(
                blk, ((dj2 * C1, K2 - HC2 * C1 - dj2 * C1),
                      (0, N2 - HC2 * C2)))
        w2_rows.append(acc)
    w2 = jnp.stack(w2_rows).astype(bf16)

    f1r = f1w.reshape(f1w.shape[0], C2, HP2, HP2).astype(f32)
    f1_rows = []
    for pi2 in range(HP2):
        A = jnp.transpose(f1r[:, :, pi2, :], (2, 1, 0))
        A = jnp.pad(A, ((0, 0), (0, 0), (0, LANE - A.shape[2])))
        Z = jnp.stack([A, jnp.zeros_like(A)], axis=1)
        f1_rows.append(Z.reshape(HP2 * 2 * C2, LANE)[:KF])
    f1 = jnp.stack(f1_rows).astype(bf16)

    f2 = jnp.pad(f2w.T.astype(f32),
                 ((0, LANE - f2w.shape[1]), (0, LANE - f2w.shape[0])))

    b1 = jnp.tile(c1b.astype(f32), HP)
    b1 = jnp.pad(b1, (0, N1H - b1.shape[0])).reshape(1, N1H)
    b2 = jnp.tile(c2b.astype(f32), HC2)
    b2 = jnp.pad(b2, (0, N2 - b2.shape[0])).reshape(1, N2)
    fb1 = jnp.pad(f1b.astype(f32), (0, LANE - f1b.shape[0])).reshape(1, LANE)
    fb2 = jnp.pad(f2b.astype(f32), (0, LANE - f2b.shape[0])).reshape(1, LANE)
    return w1, w2, f1, f2, b1, b2, fb1, fb2


def kernel(x, conv1_w, conv1_b, conv2_w, conv2_b, fc1_w, fc1_b, fc2_w, fc2_b):
    n = x.shape[0]
    num_outputs = fc2_w.shape[0]
    f32, bf16 = jnp.float32, jnp.bfloat16

    B = next((b for b in (128, 64, 32, 16, 8) if n % b == 0), n)
    R = B * H
    R2 = B * (H // 2)
    M2 = R2 - 2

    xr = x.reshape(n * H, W)
    w1, w2, f1, f2, b1, b2, fb1, fb2 = _prep_weights(
        conv1_w, conv1_b, conv2_w, conv2_b, fc1_w, fc1_b, fc2_w, fc2_b)

    def body(x_ref, w1_ref, w2_ref, f1_ref, f2_ref,
             b1_ref, b2_ref, fb1_ref, fb2_ref, out_ref,
             lhs1, ye0, ye1, yo0, yo1, p1, sa, sb, sc):
        y1 = (ye0, ye1, yo0, yo1)
        s2 = (sa, sb, sc)
        lhs1[...] = jnp.zeros_like(lhs1)
        for di in range(3):
            lhs1[pl.ds(0, R - di), di * XW: di * XW + XW] = (
                x_ref[pl.ds(di, R - di), :].astype(bf16))

        lhs = lhs1[...]
        for q in range(4):
            y1[q][...] = jnp.dot(lhs, w1_ref[:, q * 128:(q + 1) * 128],
                                 preferred_element_type=f32)

        for q in range(2):
            ye, yo = y1[q], y1[q + 2]
            m = jnp.maximum(
                jnp.maximum(ye[pl.ds(0, R2, 2), :], ye[pl.ds(1, R2, 2), :]),
                jnp.maximum(yo[pl.ds(0, R2, 2), :], yo[pl.ds(1, R2, 2), :]))
            p1[:, q * 128:(q + 1) * 128] = jnp.maximum(
                m + b1_ref[:, q * 128:(q + 1) * 128], 0.0).astype(bf16)

        acc = None
        for di2 in range(3):
            lhs2 = p1[pl.ds(di2, M2), :]
            prod = jnp.dot(lhs2, w2_ref[di2], preferred_element_type=f32)
            acc = prod if acc is None else acc + prod
        acc = jnp.maximum(acc + b2_ref[...], 0.0)
        for q in range(3):
            s2[q][pl.ds(0, M2), :] = acc[:, q * 128:(q + 1) * 128]

        hacc = None
        for pi2 in range(HP2):
            vm = jnp.concatenate(
                [jnp.maximum(s2[q][pl.ds(2 * pi2, B, 14), :],
                             s2[q][pl.ds(2 * pi2 + 1, B, 14), :])
                 for q in range(3)], axis=1)
            vmm = jnp.maximum(vm[:, 0:KF], vm[:, C2:KF + C2])
            prod = jnp.dot(vmm.astype(bf16), f1_ref[pi2],
                           preferred_element_type=f32)
            hacc = prod if hacc is None else hacc + prod
        hidden = jnp.maximum(hacc + fb1_ref[...], 0.0)

        logits = jnp.dot(hidden, f2_ref[...],
                         preferred_element_type=f32) + fb2_ref[...]
        lane = lax.broadcasted_iota(jnp.int32, (B, LANE), 1)
        logits = jnp.where(lane < num_outputs, logits, -1e30)
        mx = jnp.max(logits, axis=-1, keepdims=True)
        lse = jnp.log(jnp.sum(jnp.exp(logits - mx), axis=-1, keepdims=True))
        out_ref[...] = (logits - mx - lse)[:, :num_outputs]

    grid_spec = pltpu.PrefetchScalarGridSpec(
        num_scalar_prefetch=0,
        grid=(n // B,),
        in_specs=[
            pl.BlockSpec((R, XW), lambda g: (g, 0)),
            pl.BlockSpec((K1, 2 * N1H), lambda g: (0, 0)),
            pl.BlockSpec((3, K2, N2), lambda g: (0, 0, 0)),
            pl.BlockSpec((HP2, KF, LANE), lambda g: (0, 0, 0)),
            pl.BlockSpec((LANE, LANE), lambda g: (0, 0)),
            pl.BlockSpec((1, N1H), lambda g: (0, 0)),
            pl.BlockSpec((1, N2), lambda g: (0, 0)),
            pl.BlockSpec((1, LANE), lambda g: (0, 0)),
            pl.BlockSpec((1, LANE), lambda g: (0, 0)),
        ],
        out_specs=pl.BlockSpec((B, num_outputs), lambda g: (g, 0)),
        scratch_shapes=[
            pltpu.VMEM((R, K1), bf16),
            pltpu.VMEM((R, 128), f32),
            pltpu.VMEM((R, 128), f32),
            pltpu.VMEM((R, 128), f32),
            pltpu.VMEM((R, 128), f32),
            pltpu.VMEM((R2, K2), bf16),
            pltpu.VMEM((R2, 128), f32),
            pltpu.VMEM((R2, 128), f32),
            pltpu.VMEM((R2, 128), f32),
        ],
    )

    out = pl.pallas_call(
        body,
        out_shape=jax.ShapeDtypeStruct((n, num_outputs), f32),
        grid_spec=grid_spec,
        compiler_params=pltpu.CompilerParams(
            dimension_semantics=("parallel",)),
    )(xr, w1, w2, f1, f2, b1, b2, fb1, fb2)
    return out
